# baseline (speedup 1.0000x reference)
import functools

import jax
import jax.numpy as jnp
from jax import lax
from jax.experimental import pallas as pl
from jax.experimental.pallas import tpu as pltpu
from jax.experimental.pallas import tpu_sc as plsc

NC = 2
NS = 16
NW = NC * NS
CHUNK = 128
NBUF = 8


def _gather_kernel(n_chunks, n_per_w, D, src_hbm, table_hbm, out_hbm,
                   idx_v, bufs, gsems, ssems):
    wid = lax.axis_index("s") * NC + lax.axis_index("c")
    base = wid * n_per_w
    pltpu.sync_copy(src_hbm.at[wid], idx_v)

    for b in range(NBUF):
        pltpu.async_copy(table_hbm.at[idx_v.at[b]], bufs[b], gsems[b])

    @pl.loop(0, n_chunks, step=NBUF)
    def _(g):
        for b in range(NBUF):
            j = g + b
            pltpu.make_async_copy(table_hbm.at[idx_v.at[j]], bufs[b],
                                  gsems[b]).wait()
            out_slc = out_hbm.at[pl.ds(base + j * CHUNK, CHUNK)]
            pltpu.async_copy(bufs[b], out_slc, ssems[b])
            nj = j + NBUF

            @pl.when(nj < n_chunks)
            def _():
                pltpu.make_async_copy(bufs[b], out_slc, ssems[b]).wait()
                pltpu.async_copy(table_hbm.at[idx_v.at[nj]], bufs[b],
                                 gsems[b])

    for b in range(NBUF):
        out_slc = out_hbm.at[pl.ds(base, CHUNK)]
        pltpu.make_async_copy(bufs[b], out_slc, ssems[b]).wait()


def kernel(src, table):
    B, S = src.shape
    V, D = table.shape
    N = B * S
    n_per_w = N // NW
    n_chunks = n_per_w // CHUNK
    idx = src.reshape(NW, n_chunks, CHUNK)

    mesh = plsc.VectorSubcoreMesh(core_axis_name="c", subcore_axis_name="s")
    run = functools.partial(
        pl.kernel,
        out_type=jax.ShapeDtypeStruct((N, D), jnp.float32),
        mesh=mesh,
        scratch_types=[
            pltpu.VMEM((n_chunks, CHUNK), jnp.int32),
            [pltpu.VMEM((CHUNK, D), jnp.float32) for _ in range(NBUF)],
            [pltpu.SemaphoreType.DMA for _ in range(NBUF)],
            [pltpu.SemaphoreType.DMA for _ in range(NBUF)],
        ],
        compiler_params=pltpu.CompilerParams(use_tc_tiling_on_sc=False),
    )(functools.partial(_gather_kernel, n_chunks, n_per_w, D))
    out = run(idx, table)
    return out.reshape(B, S, D)
